# baseline (speedup 1.0000x reference)
import functools

import jax
import jax.numpy as jnp
from jax import lax
from jax.experimental import pallas as pl
from jax.experimental.pallas import tpu as pltpu
from jax.experimental.pallas import tpu_sc as plsc

BATCH = 16384
FIELDS = 26
EMBEDDING_SIZE = 32
CARD = 1000000

NC = 2
NS = 16
NW = NC * NS

B = BATCH * FIELDS
D = EMBEDDING_SIZE
W4 = CARD // 4
BPW = B // NW
CHUNK = 128
NCHUNK = BPW // CHUNK
NBUF = 4
L = 16
GRP = CHUNK // L
assert BPW * NW == B and NCHUNK * CHUNK == BPW and (NCHUNK - 4) % 2 == 0


def _gather_kernel(idx_hbm, w4_hbm, out_hbm, idx_v, g_v, rows4_v, outq_v,
                   gsem0, gsem1, osem0, osem1):
    wid = lax.axis_index("s") * NC + lax.axis_index("c")
    pltpu.sync_copy(idx_hbm.at[wid], idx_v)

    def gbody(t, _):
        j = t // GRP
        o = lax.rem(t, GRP) * L
        g_v[j, pl.ds(o, L)] = lax.shift_right_logical(idx_v[j, pl.ds(o, L)], 2)
        return 0

    lax.fori_loop(0, NCHUNK * GRP, gbody, 0)

    gsems = (gsem0, gsem1)
    osems = (osem0, osem1)

    def gather_chunk(j, slot, par):
        return pltpu.make_async_copy(
            w4_hbm.at[g_v.at[j]], rows4_v.at[slot], gsems[par])

    def out_chunk(j, eslot, par):
        return pltpu.make_async_copy(
            outq_v.at[eslot], out_hbm.at[wid, j], osems[par])

    lanes = lax.iota(jnp.int32, L)

    def extract(j, slot, eslot):
        slot_v = jnp.full((L,), slot, jnp.int32)
        eslot_v = jnp.full((L,), eslot, jnp.int32)

        def grp(t, _):
            o = t * L
            r = idx_v[j, pl.ds(o, L)]
            col0 = lax.bitwise_and(r, 3) * D
            row = lanes + o
            for k in range(D):
                vals = plsc.load_gather(rows4_v, [slot_v, row, col0 + k])
                plsc.store_scatter(
                    outq_v, [eslot_v, row, jnp.full((L,), k, jnp.int32)], vals)
            return 0

        lax.fori_loop(0, GRP, grp, 0)

    gather_chunk(0, 0, 0).start()
    gather_chunk(1, 1, 1).start()

    for j in (0, 1):
        gather_chunk(j, j, j % 2).wait()
        gather_chunk(j + 2, j + 2, j % 2).start()
        extract(j, j, j % 2)
        out_chunk(j, j % 2, j % 2).start()

    def step(j, par):
        slot = lax.rem(j, NBUF)
        gather_chunk(j, slot, par).wait()
        out_chunk(j - 2, par, par).wait()
        gather_chunk(j + 2, lax.rem(j + 2, NBUF), par).start()
        extract(j, slot, par)
        out_chunk(j, par, par).start()

    def body(i, _):
        j = 2 + 2 * i
        step(j, 0)
        step(j + 1, 1)
        return 0

    lax.fori_loop(0, (NCHUNK - 4) // 2, body, 0)

    for j in (NCHUNK - 2, NCHUNK - 1):
        gather_chunk(j, j % NBUF, j % 2).wait()
        out_chunk(j - 2, j % 2, j % 2).wait()
        extract(j, j % NBUF, j % 2)
        out_chunk(j, j % 2, j % 2).start()
    for j in (NCHUNK - 2, NCHUNK - 1):
        out_chunk(j, j % 2, j % 2).wait()


@jax.jit
def kernel(inputs, weight):
    idx = inputs.astype(jnp.int32).reshape(NW, NCHUNK, CHUNK)
    w4 = weight.reshape(W4, 4 * D)
    mesh = plsc.VectorSubcoreMesh(core_axis_name="c", subcore_axis_name="s")
    out = pl.kernel(
        _gather_kernel,
        out_type=jax.ShapeDtypeStruct((NW, NCHUNK, CHUNK, D), jnp.float32),
        mesh=mesh,
        scratch_types=[
            pltpu.VMEM((NCHUNK, CHUNK), jnp.int32),
            pltpu.VMEM((NCHUNK, CHUNK), jnp.int32),
            pltpu.VMEM((NBUF, CHUNK, 4 * D), jnp.float32),
            pltpu.VMEM((2, CHUNK, D), jnp.float32),
            pltpu.SemaphoreType.DMA,
            pltpu.SemaphoreType.DMA,
            pltpu.SemaphoreType.DMA,
            pltpu.SemaphoreType.DMA,
        ],
        compiler_params=pltpu.CompilerParams(needs_layout_passes=False),
    )(idx, w4)
    return out.reshape(BATCH, FIELDS, D)


# speedup vs baseline: 1.5700x; 1.5700x over previous
import functools

import jax
import jax.numpy as jnp
from jax import lax
from jax.experimental import pallas as pl
from jax.experimental.pallas import tpu as pltpu
from jax.experimental.pallas import tpu_sc as plsc

BATCH = 16384
FIELDS = 26
EMBEDDING_SIZE = 32
CARD = 1000000

NC = 2
NS = 16
NW = NC * NS

D = EMBEDDING_SIZE
W4 = CARD // 4
BPW = BATCH // NW
CB = 128
NCB = BPW // CB
NCHUNK = FIELDS * NCB
NBUF = 4
L = 16
assert BPW * NW == BATCH and (NCHUNK - 4) % 2 == 0


def _gather_kernel(idxt_hbm, w4_hbm, out_hbm, idxt_v, g_v, rows4_v, fbuf_v,
                   gsem0, gsem1, osem0, osem1):
    wid = lax.axis_index("s") * NC + lax.axis_index("c")
    b0 = wid * BPW
    pltpu.sync_copy(idxt_hbm.at[:, pl.ds(b0, BPW)], idxt_v)

    gsems = (gsem0, gsem1)
    osems = (osem0, osem1)

    def prep_gather(j, slot):
        f = j // NCB
        cb = lax.rem(j, NCB)
        for t in range(CB // L):
            g_v[slot, pl.ds(t * L, L)] = lax.shift_right_logical(
                idxt_v[f, pl.ds(cb * CB + t * L, L)], 2)

    def gather_chunk(slot, par):
        return pltpu.make_async_copy(
            w4_hbm.at[g_v.at[slot]], rows4_v.at[slot], gsems[par])

    def out_chunk(j, eslot, par):
        f = j // NCB
        cb = lax.rem(j, NCB)
        return pltpu.make_async_copy(
            fbuf_v.at[eslot],
            out_hbm.at[f, :, pl.ds(b0 + cb * CB, CB)], osems[par])

    lanes = lax.iota(jnp.int32, L)

    def extract(j, slot, eslot):
        f = j // NCB
        cb = lax.rem(j, NCB)
        slot_v = jnp.full((L,), slot, jnp.int32)

        def grp(t, _):
            o = t * L
            r = idxt_v[f, pl.ds(cb * CB + o, L)]
            col0 = lax.bitwise_and(r, 3) * D
            row = lanes + o
            for e in range(D):
                vals = plsc.load_gather(rows4_v, [slot_v, row, col0 + e])
                fbuf_v[eslot, e, pl.ds(o, L)] = vals
            return 0

        lax.fori_loop(0, CB // L, grp, 0)

    for j in (0, 1):
        prep_gather(j, j)
        gather_chunk(j, j).start()

    for j in (0, 1):
        gather_chunk(j, j % 2).wait()
        prep_gather(j + 2, j + 2)
        gather_chunk(j + 2, j % 2).start()
        extract(j, j, j % 2)
        out_chunk(j, j % 2, j % 2).start()

    def step(j, par):
        slot = lax.rem(j, NBUF)
        gather_chunk(slot, par).wait()
        out_chunk(j - 2, par, par).wait()
        nslot = lax.rem(j + 2, NBUF)
        prep_gather(j + 2, nslot)
        gather_chunk(nslot, par).start()
        extract(j, slot, par)
        out_chunk(j, par, par).start()

    def body(i, _):
        j = 2 + 2 * i
        step(j, 0)
        step(j + 1, 1)
        return 0

    lax.fori_loop(0, (NCHUNK - 4) // 2, body, 0)

    for j in (NCHUNK - 2, NCHUNK - 1):
        gather_chunk(j % NBUF, j % 2).wait()
        out_chunk(j - 2, j % 2, j % 2).wait()
        extract(j, j % NBUF, j % 2)
        out_chunk(j, j % 2, j % 2).start()
    for j in (NCHUNK - 2, NCHUNK - 1):
        out_chunk(j, j % 2, j % 2).wait()


@jax.jit
def kernel(inputs, weight):
    idxt = inputs.astype(jnp.int32).T
    w4 = weight.reshape(W4, 4 * D)
    mesh = plsc.VectorSubcoreMesh(core_axis_name="c", subcore_axis_name="s")
    out = pl.kernel(
        _gather_kernel,
        out_type=jax.ShapeDtypeStruct((FIELDS, D, BATCH), jnp.float32),
        mesh=mesh,
        scratch_types=[
            pltpu.VMEM((FIELDS, BPW), jnp.int32),
            pltpu.VMEM((NBUF, CB), jnp.int32),
            pltpu.VMEM((NBUF, CB, 4 * D), jnp.float32),
            pltpu.VMEM((2, D, CB), jnp.float32),
            pltpu.SemaphoreType.DMA,
            pltpu.SemaphoreType.DMA,
            pltpu.SemaphoreType.DMA,
            pltpu.SemaphoreType.DMA,
        ],
        compiler_params=pltpu.CompilerParams(needs_layout_passes=False),
    )(idxt, w4)
    return out.transpose(2, 0, 1)


# speedup vs baseline: 1.7340x; 1.1044x over previous
import functools

import jax
import jax.numpy as jnp
from jax import lax
from jax.experimental import pallas as pl
from jax.experimental.pallas import tpu as pltpu
from jax.experimental.pallas import tpu_sc as plsc

BATCH = 16384
FIELDS = 26
EMBEDDING_SIZE = 32
CARD = 1000000

NC = 2
NS = 16
NW = NC * NS

D = EMBEDDING_SIZE
BPW = BATCH // NW
CB = 128
NCB = BPW // CB
NCHUNK = FIELDS * NCB
NBUF = 4
assert BPW * NW == BATCH and (NCHUNK - 4) % 2 == 0


def _gather_kernel(idxt_hbm, w_hbm, out_hbm, idxt_v, rows_v,
                   gsem0, gsem1, osem0, osem1):
    wid = lax.axis_index("s") * NC + lax.axis_index("c")
    b0 = wid * BPW
    pltpu.sync_copy(idxt_hbm.at[:, pl.ds(b0, BPW)], idxt_v)

    gsems = (gsem0, gsem1)
    osems = (osem0, osem1)

    def gather_chunk(j, slot, par):
        f = j // NCB
        cb = lax.rem(j, NCB)
        return pltpu.make_async_copy(
            w_hbm.at[idxt_v.at[f, pl.ds(cb * CB, CB)]],
            rows_v.at[slot], gsems[par])

    def out_chunk(j, slot, par):
        f = j // NCB
        cb = lax.rem(j, NCB)
        return pltpu.make_async_copy(
            rows_v.at[slot],
            out_hbm.at[f, pl.ds(b0 + cb * CB, CB)], osems[par])

    gather_chunk(0, 0, 0).start()
    gather_chunk(1, 1, 1).start()

    for j in (0, 1):
        gather_chunk(j, j, j % 2).wait()
        gather_chunk(j + 2, j + 2, j % 2).start()
        out_chunk(j, j, j % 2).start()

    def step(j, par):
        slot = lax.rem(j, NBUF)
        gather_chunk(j, slot, par).wait()
        out_chunk(j - 2, lax.rem(j - 2, NBUF), par).wait()
        gather_chunk(j + 2, lax.rem(j + 2, NBUF), par).start()
        out_chunk(j, slot, par).start()

    def body(i, _):
        j = 2 + 2 * i
        step(j, 0)
        step(j + 1, 1)
        return 0

    lax.fori_loop(0, (NCHUNK - 4) // 2, body, 0)

    for j in (NCHUNK - 2, NCHUNK - 1):
        gather_chunk(j, j % NBUF, j % 2).wait()
        out_chunk(j - 2, (j - 2) % NBUF, j % 2).wait()
        out_chunk(j, j % NBUF, j % 2).start()
    for j in (NCHUNK - 2, NCHUNK - 1):
        out_chunk(j, j % NBUF, j % 2).wait()


@jax.jit
def kernel(inputs, weight):
    idxt = inputs.astype(jnp.int32).T
    mesh = plsc.VectorSubcoreMesh(core_axis_name="c", subcore_axis_name="s")
    out = pl.kernel(
        _gather_kernel,
        out_type=jax.ShapeDtypeStruct((FIELDS, BATCH, D), jnp.float32),
        mesh=mesh,
        scratch_types=[
            pltpu.VMEM((FIELDS, BPW), jnp.int32),
            pltpu.VMEM((NBUF, CB, D), jnp.float32),
            pltpu.SemaphoreType.DMA,
            pltpu.SemaphoreType.DMA,
            pltpu.SemaphoreType.DMA,
            pltpu.SemaphoreType.DMA,
        ],
        compiler_params=pltpu.CompilerParams(use_tc_tiling_on_sc=False),
    )(idxt, weight)
    return out.transpose(1, 0, 2)
